# baseline (speedup 1.0000x reference)
import functools

import jax
import jax.numpy as jnp
from jax import lax
from jax.experimental import pallas as pl
from jax.experimental.pallas import tpu as pltpu

_ALPHA = 0.25
_BB = 32


def _body(xt_ref, sit_ref, tlt_ref, pbt_ref, tbt_ref,
          ce_ref, l1_ref, gi_ref, cls_ref, *, inv_nb):
    g = pl.program_id(0)
    C, BB, Q = xt_ref.shape
    T, B = sit_ref.shape

    @pl.when(g == 0)
    def _first():
        ce_ref[...] = jnp.zeros((1, 1), jnp.float32)
        sit = sit_ref[...]
        tlt = tlt_ref[...]
        qio = lax.broadcasted_iota(jnp.int32, (T, B, Q), 2)
        eq = sit[:, :, None] == qio
        tio2 = lax.broadcasted_iota(jnp.int32, (T, B), 0)
        key = tio2 * 128 + tlt
        kmax = jnp.max(jnp.where(eq, key[:, :, None], -1), axis=0)
        cls_ref[...] = jnp.where(kmax >= 0, kmax & 127, C)

        sel = eq.astype(jnp.float32)

        def coord(k):
            src = jnp.sum(sel * pbt_ref[:, k, :][None], axis=2)
            return src, tbt_ref[:, k, :]

        scx, tcx = coord(0)
        scy, tcy = coord(1)
        sw, tw = coord(2)
        sh, th = coord(3)
        l1 = (jnp.abs(scx - tcx) + jnp.abs(scy - tcy)
              + jnp.abs(sw - tw) + jnp.abs(sh - th))
        l1_ref[...] = (jnp.sum(l1) * inv_nb).reshape(1, 1)

        sx0, sx1 = scx - 0.5 * sw, scx + 0.5 * sw
        sy0, sy1 = scy - 0.5 * sh, scy + 0.5 * sh
        tx0, tx1 = tcx - 0.5 * tw, tcx + 0.5 * tw
        ty0, ty1 = tcy - 0.5 * th, tcy + 0.5 * th
        a1 = (sx1 - sx0) * (sy1 - sy0)
        a2 = (tx1 - tx0) * (ty1 - ty0)
        iw = jnp.maximum(jnp.minimum(sx1, tx1) - jnp.maximum(sx0, tx0), 0.0)
        ih = jnp.maximum(jnp.minimum(sy1, ty1) - jnp.maximum(sy0, ty0), 0.0)
        inter = iw * ih
        union = a1 + a2 - inter
        iou = inter / union
        ew = jnp.maximum(sx1, tx1) - jnp.minimum(sx0, tx0)
        eh = jnp.maximum(sy1, ty1) - jnp.minimum(sy0, ty0)
        ae = ew * eh
        giou = iou - (ae - union) / ae
        gi_ref[...] = (jnp.sum(1.0 - giou) * inv_nb).reshape(1, 1)

    x = xt_ref[...]
    cls_blk = cls_ref[pl.ds(pl.multiple_of(g * BB, BB), BB), :]
    cio = lax.broadcasted_iota(jnp.int32, (C, BB, Q), 0)
    mb = cio == cls_blk[None]
    e = jnp.exp(-jnp.abs(x))
    d = 1.0 + e
    l = jnp.log(d)
    sp = jnp.maximum(x, 0.0) + l
    r = 1.0 / d
    p = jnp.where(x >= 0.0, r, 1.0 - r)
    acc0 = jnp.sum(p * (p * sp))
    xm = jnp.sum(jnp.where(mb, x, 0.0), axis=0)

    em = jnp.exp(-jnp.abs(xm))
    dm = 1.0 + em
    lm = jnp.log(dm)
    spm = jnp.maximum(xm, 0.0) + lm
    rm = 1.0 / dm
    pm = jnp.where(xm >= 0.0, rm, 1.0 - rm)
    omm = 1.0 - pm
    delta = (_ALPHA * omm * omm * (spm - xm)
             - (1.0 - _ALPHA) * pm * pm * spm)
    delta = jnp.where(cls_blk < C, delta, 0.0)
    ce_ref[...] += (((1.0 - _ALPHA) * acc0 + jnp.sum(delta))
                    * inv_nb).reshape(1, 1)


def kernel(pred_logits, pred_boxes, tgt_boxes, src_idx, tgt_labels):
    B, Q, C = pred_logits.shape
    T = src_idx.shape[1]
    f32 = jnp.float32
    inv_nb = 1.0 / float(B * T)

    xt = jnp.transpose(pred_logits, (2, 0, 1))
    sit = jnp.transpose(src_idx, (1, 0))
    tlt = jnp.transpose(tgt_labels, (1, 0))
    pbt = jnp.transpose(pred_boxes, (0, 2, 1))
    tbt = jnp.transpose(tgt_boxes, (1, 2, 0))

    ce, l1, gi = pl.pallas_call(
        functools.partial(_body, inv_nb=inv_nb),
        grid=(B // _BB,),
        in_specs=[
            pl.BlockSpec((C, _BB, Q), lambda g: (0, g, 0)),
            pl.BlockSpec((T, B), lambda g: (0, 0)),
            pl.BlockSpec((T, B), lambda g: (0, 0)),
            pl.BlockSpec((B, 4, Q), lambda g: (0, 0, 0)),
            pl.BlockSpec((T, 4, B), lambda g: (0, 0, 0)),
        ],
        out_specs=[pl.BlockSpec((1, 1), lambda g: (0, 0))] * 3,
        out_shape=[jax.ShapeDtypeStruct((1, 1), f32)] * 3,
        scratch_shapes=[pltpu.VMEM((B, Q), jnp.int32)],
    )(xt, sit, tlt, pbt, tbt)

    return (ce[0, 0], l1[0, 0], gi[0, 0])


# speedup vs baseline: 1.0229x; 1.0229x over previous
import functools

import jax
import jax.numpy as jnp
from jax import lax
from jax.experimental import pallas as pl
from jax.experimental.pallas import tpu as pltpu

_ALPHA = 0.25
_BB = 16


def _body(xt_ref, sit_ref, tlt_ref, pbt_ref, tbt_ref,
          ce_ref, l1_ref, gi_ref, cls_ref, *, inv_nb):
    g = pl.program_id(0)
    C, BB, Q = xt_ref.shape
    T, B = sit_ref.shape

    @pl.when(g == 0)
    def _first():
        ce_ref[...] = jnp.zeros((1, 1), jnp.float32)
        sit = sit_ref[...]
        tlt = tlt_ref[...]
        qio = lax.broadcasted_iota(jnp.int32, (T, B, Q), 2)
        eq = sit[:, :, None] == qio
        tio2 = lax.broadcasted_iota(jnp.int32, (T, B), 0)
        key = tio2 * 128 + tlt
        kmax = jnp.max(jnp.where(eq, key[:, :, None], -1), axis=0)
        cls_ref[...] = jnp.where(kmax >= 0, kmax & 127, C)

        sel = eq.astype(jnp.float32)

        def coord(k):
            src = jnp.sum(sel * pbt_ref[:, k, :][None], axis=2)
            return src, tbt_ref[:, k, :]

        scx, tcx = coord(0)
        scy, tcy = coord(1)
        sw, tw = coord(2)
        sh, th = coord(3)
        l1 = (jnp.abs(scx - tcx) + jnp.abs(scy - tcy)
              + jnp.abs(sw - tw) + jnp.abs(sh - th))
        l1_ref[...] = (jnp.sum(l1) * inv_nb).reshape(1, 1)

        sx0, sx1 = scx - 0.5 * sw, scx + 0.5 * sw
        sy0, sy1 = scy - 0.5 * sh, scy + 0.5 * sh
        tx0, tx1 = tcx - 0.5 * tw, tcx + 0.5 * tw
        ty0, ty1 = tcy - 0.5 * th, tcy + 0.5 * th
        a1 = (sx1 - sx0) * (sy1 - sy0)
        a2 = (tx1 - tx0) * (ty1 - ty0)
        iw = jnp.maximum(jnp.minimum(sx1, tx1) - jnp.maximum(sx0, tx0), 0.0)
        ih = jnp.maximum(jnp.minimum(sy1, ty1) - jnp.maximum(sy0, ty0), 0.0)
        inter = iw * ih
        union = a1 + a2 - inter
        iou = inter / union
        ew = jnp.maximum(sx1, tx1) - jnp.minimum(sx0, tx0)
        eh = jnp.maximum(sy1, ty1) - jnp.minimum(sy0, ty0)
        ae = ew * eh
        giou = iou - (ae - union) / ae
        gi_ref[...] = (jnp.sum(1.0 - giou) * inv_nb).reshape(1, 1)

    x = xt_ref[...]
    cls_blk = cls_ref[pl.ds(pl.multiple_of(g * BB, BB), BB), :]
    cio = lax.broadcasted_iota(jnp.int32, (C, BB, Q), 0)
    mb = cio == cls_blk[None]
    e = jnp.exp(-jnp.abs(x))
    d = 1.0 + e
    l = jnp.log(d)
    sp = jnp.maximum(x, 0.0) + l
    p2 = jnp.exp(2.0 * (x - sp))
    acc0 = jnp.sum(p2 * sp)
    xm = jnp.sum(jnp.where(mb, x, 0.0), axis=0)

    em = jnp.exp(-jnp.abs(xm))
    dm = 1.0 + em
    lm = jnp.log(dm)
    spm = jnp.maximum(xm, 0.0) + lm
    rm = 1.0 / dm
    pm = jnp.where(xm >= 0.0, rm, 1.0 - rm)
    omm = 1.0 - pm
    delta = (_ALPHA * omm * omm * (spm - xm)
             - (1.0 - _ALPHA) * pm * pm * spm)
    delta = jnp.where(cls_blk < C, delta, 0.0)
    ce_ref[...] += (((1.0 - _ALPHA) * acc0 + jnp.sum(delta))
                    * inv_nb).reshape(1, 1)


def kernel(pred_logits, pred_boxes, tgt_boxes, src_idx, tgt_labels):
    B, Q, C = pred_logits.shape
    T = src_idx.shape[1]
    f32 = jnp.float32
    inv_nb = 1.0 / float(B * T)

    xt = jnp.transpose(pred_logits, (2, 0, 1))
    sit = jnp.transpose(src_idx, (1, 0))
    tlt = jnp.transpose(tgt_labels, (1, 0))
    pbt = jnp.transpose(pred_boxes, (0, 2, 1))
    tbt = jnp.transpose(tgt_boxes, (1, 2, 0))

    ce, l1, gi = pl.pallas_call(
        functools.partial(_body, inv_nb=inv_nb),
        grid=(B // _BB,),
        in_specs=[
            pl.BlockSpec((C, _BB, Q), lambda g: (0, g, 0)),
            pl.BlockSpec((T, B), lambda g: (0, 0)),
            pl.BlockSpec((T, B), lambda g: (0, 0)),
            pl.BlockSpec((B, 4, Q), lambda g: (0, 0, 0)),
            pl.BlockSpec((T, 4, B), lambda g: (0, 0, 0)),
        ],
        out_specs=[pl.BlockSpec((1, 1), lambda g: (0, 0))] * 3,
        out_shape=[jax.ShapeDtypeStruct((1, 1), f32)] * 3,
        scratch_shapes=[pltpu.VMEM((B, Q), jnp.int32)],
    )(xt, sit, tlt, pbt, tbt)

    return (ce[0, 0], l1[0, 0], gi[0, 0])
